# baseline (speedup 1.0000x reference)
import functools

import jax
import jax.numpy as jnp
from jax import lax
from jax.experimental import pallas as pl
from jax.experimental.pallas import tpu as pltpu
from jax.experimental.pallas import tpu_sc as plsc


def _gather_rows_sc(table, ind_flat, n_per_batch, rows_per_batch):
    width = table.shape[1]
    nrows = ind_flat.shape[0]

    info = plsc.get_sparse_core_info()
    num_cores, num_subcores, lanes = info.num_cores, info.num_subcores, info.num_lanes
    num_workers = num_cores * num_subcores
    per = nrows // num_workers

    mesh = plsc.VectorSubcoreMesh(core_axis_name="c", subcore_axis_name="s")

    @functools.partial(
        pl.kernel,
        mesh=mesh,
        out_type=jax.ShapeDtypeStruct((nrows, width), jnp.float32),
        scratch_types=[
            pltpu.VMEM((per,), jnp.int32),
            pltpu.VMEM((per, width), jnp.float32),
            pltpu.SemaphoreType.DMA,
        ],
    )
    def gather_kernel(table_hbm, idx_hbm, pred_hbm, idx_v, rows_v, sem):
        wid = lax.axis_index("s") * num_cores + lax.axis_index("c")
        base = wid * per
        row_base = (base // n_per_batch) * rows_per_batch
        pltpu.sync_copy(idx_hbm.at[pl.ds(base, per)], idx_v)
        for j in range(per // lanes):
            sl = pl.ds(j * lanes, lanes)
            idx_v[sl] = (idx_v[sl] >> 1) + row_base
        pltpu.async_copy(table_hbm.at[idx_v], rows_v, sem).wait()
        pltpu.sync_copy(rows_v, pred_hbm.at[pl.ds(base, per)])

    return gather_kernel(table, ind_flat)


def _masked_bce_tc(pred2, ind_col, target, maskf, win_sq):
    def body(p2_ref, i_ref, t_ref, m_ref, o_ref):
        p2 = p2_ref[...]
        odd = (i_ref[...] & 1) == 1
        p = jnp.where(odd, p2[:, win_sq:], p2[:, :win_sq])
        t = t_ref[...]
        m = m_ref[...]
        c = jnp.log1p(jnp.exp(-jnp.abs(p)))
        mn = jnp.minimum(p, 0.0)
        ls_p = mn - c
        ls_mp = mn - p - c
        bce = -(1.5 * t * ls_p + (1.0 - t) * ls_mp)
        loss_sum = jnp.sum(bce * m)
        num_sample = jnp.sum(m) * float(win_sq)
        o_ref[0, 0] = jnp.where(num_sample > 0, loss_sum / num_sample, loss_sum)

    return pl.pallas_call(
        body,
        out_shape=jax.ShapeDtypeStruct((1, 1), jnp.float32),
        out_specs=pl.BlockSpec(memory_space=pltpu.SMEM),
    )(pred2, ind_col, target, maskf)


def kernel(output, mask, ind, target):
    B, S, W = output.shape
    N = ind.shape[1]
    win_sq = target.shape[-1] * target.shape[-2]
    table = output.reshape(B * S * W // (2 * win_sq), 2 * win_sq)
    ind_flat = ind.reshape(B * N)
    pred2 = _gather_rows_sc(table, ind_flat, N, S * W // (2 * win_sq))
    tgt = target.reshape(B * N, win_sq)
    maskf = mask.reshape(B * N, 1).astype(jnp.float32)
    loss = _masked_bce_tc(pred2, ind_flat.reshape(B * N, 1), tgt, maskf, win_sq)
    return loss[0, 0]


# speedup vs baseline: 2.5713x; 2.5713x over previous
import functools

import jax
import jax.numpy as jnp
from jax import lax
from jax.experimental import pallas as pl
from jax.experimental.pallas import tpu as pltpu

_K_CHUNKS = 4


def _masked_bce_tc(p_view, ind, target_t, maskf, win_sq):
    B, W, S = p_view.shape
    N = ind.shape[-1]
    chunk = S // _K_CHUNKS

    def body(p_ref, i_ref, t_ref, m_ref, o_ref, acc_ref):
        k = pl.program_id(1)

        @pl.when(k == 0)
        def _():
            acc_ref[...] = jnp.zeros_like(acc_ref)

        indv = i_ref[0, 0]
        local = indv - k * chunk
        oh = (lax.broadcasted_iota(jnp.int32, (N, chunk), 1)
              == local[:, None]).astype(jnp.float32)
        p = p_ref[0]
        acc_ref[...] += lax.dot_general(
            p, oh, (((1,), (1,)), ((), ())),
            preferred_element_type=jnp.float32)

        @pl.when(k == _K_CHUNKS - 1)
        def _():
            pred = acc_ref[...]
            t = t_ref[0]
            m = m_ref[0]
            c = jnp.log1p(jnp.exp(-jnp.abs(pred)))
            mn = jnp.minimum(pred, 0.0)
            ls_p = mn - c
            ls_mp = mn - pred - c
            bce = -(1.5 * t * ls_p + (1.0 - t) * ls_mp)
            o_ref[0, 0, 0] = jnp.sum(bce * m)
            o_ref[0, 0, 1] = jnp.sum(m) * float(win_sq)

    grid = (B, _K_CHUNKS)
    return pl.pallas_call(
        body,
        grid=grid,
        in_specs=[
            pl.BlockSpec((1, W, chunk), lambda b, k: (b, 0, k)),
            pl.BlockSpec((1, 1, N), lambda b, k: (b, 0, 0)),
            pl.BlockSpec((1, W, N), lambda b, k: (b, 0, 0)),
            pl.BlockSpec((1, 1, N), lambda b, k: (b, 0, 0)),
        ],
        out_specs=pl.BlockSpec((1, 1, 2), lambda b, k: (b, 0, 0),
                               memory_space=pltpu.SMEM),
        out_shape=jax.ShapeDtypeStruct((B, 1, 2), jnp.float32),
        scratch_shapes=[pltpu.VMEM((W, N), jnp.float32)],
    )(p_view, ind, target_t, maskf)


def kernel(output, mask, ind, target):
    B, S, W = output.shape
    N = ind.shape[1]
    win_sq = target.shape[-1] * target.shape[-2]
    p_view = output.transpose(0, 2, 1)
    target_t = target.reshape(B, N, win_sq).transpose(0, 2, 1)
    maskf = mask.astype(jnp.float32).reshape(B, 1, N)
    parts = _masked_bce_tc(p_view, ind.reshape(B, 1, N), target_t, maskf, win_sq)
    loss_sum = jnp.sum(parts[:, 0, 0])
    num_sample = jnp.sum(parts[:, 0, 1])
    return jnp.where(num_sample > 0, loss_sum / num_sample, loss_sum)


# speedup vs baseline: 2.5751x; 1.0015x over previous
import functools

import jax
import jax.numpy as jnp
from jax import lax
from jax.experimental import pallas as pl
from jax.experimental.pallas import tpu as pltpu

_K_CHUNKS = 4


def _masked_bce_tc(p_view, ind, target_t, maskf, win_sq):
    B, W, S = p_view.shape
    N = ind.shape[-1]
    chunk = S // _K_CHUNKS

    def body(p_ref, i_ref, t_ref, m_ref, o_ref, acc_ref):
        k = pl.program_id(1)

        @pl.when(k == 0)
        def _():
            acc_ref[...] = jnp.zeros_like(acc_ref)

        indv = i_ref[0, 0]
        local = indv - k * chunk
        oh = (lax.broadcasted_iota(jnp.int32, (N, chunk), 1)
              == local[:, None]).astype(jnp.bfloat16)
        p = p_ref[0].astype(jnp.bfloat16)
        acc_ref[...] += lax.dot_general(
            p, oh, (((1,), (1,)), ((), ())),
            preferred_element_type=jnp.float32)

        @pl.when(k == _K_CHUNKS - 1)
        def _():
            pred = acc_ref[...]
            t = t_ref[0]
            m = m_ref[0]
            c = jnp.log1p(jnp.exp(-jnp.abs(pred)))
            mn = jnp.minimum(pred, 0.0)
            ls_p = mn - c
            ls_mp = mn - pred - c
            bce = -(1.5 * t * ls_p + (1.0 - t) * ls_mp)
            o_ref[0, 0, 0] = jnp.sum(bce * m)
            o_ref[0, 0, 1] = jnp.sum(m) * float(win_sq)

    grid = (B, _K_CHUNKS)
    return pl.pallas_call(
        body,
        grid=grid,
        in_specs=[
            pl.BlockSpec((1, W, chunk), lambda b, k: (b, 0, k)),
            pl.BlockSpec((1, 1, N), lambda b, k: (b, 0, 0)),
            pl.BlockSpec((1, W, N), lambda b, k: (b, 0, 0)),
            pl.BlockSpec((1, 1, N), lambda b, k: (b, 0, 0)),
        ],
        out_specs=pl.BlockSpec((1, 1, 2), lambda b, k: (b, 0, 0),
                               memory_space=pltpu.SMEM),
        out_shape=jax.ShapeDtypeStruct((B, 1, 2), jnp.float32),
        scratch_shapes=[pltpu.VMEM((W, N), jnp.float32)],
    )(p_view, ind, target_t, maskf)


def kernel(output, mask, ind, target):
    B, S, W = output.shape
    N = ind.shape[1]
    win_sq = target.shape[-1] * target.shape[-2]
    p_view = output.transpose(0, 2, 1)
    target_t = target.reshape(B, N, win_sq).transpose(0, 2, 1)
    maskf = mask.astype(jnp.float32).reshape(B, 1, N)
    parts = _masked_bce_tc(p_view, ind.reshape(B, 1, N), target_t, maskf, win_sq)
    loss_sum = jnp.sum(parts[:, 0, 0])
    num_sample = jnp.sum(parts[:, 0, 1])
    return jnp.where(num_sample > 0, loss_sum / num_sample, loss_sum)


# speedup vs baseline: 4.3812x; 1.7014x over previous
import functools

import jax
import jax.numpy as jnp
from jax import lax
from jax.experimental import pallas as pl
from jax.experimental.pallas import tpu as pltpu

_K_CHUNKS = 1


def _masked_bce_tc(p_view, ind, target_t, maskf, win_sq):
    B, W, S = p_view.shape
    N = ind.shape[-1]
    chunk = S // _K_CHUNKS

    def body(p_ref, i_ref, t_ref, m_ref, o_ref, acc_ref):
        k = pl.program_id(1)

        @pl.when(k == 0)
        def _():
            acc_ref[...] = jnp.zeros_like(acc_ref)

        indv = i_ref[0, 0]
        local = indv - k * chunk
        oh = (lax.broadcasted_iota(jnp.int32, (N, chunk), 1)
              == local[:, None]).astype(jnp.bfloat16)
        p = p_ref[0].astype(jnp.bfloat16)
        acc_ref[...] += lax.dot_general(
            p, oh, (((1,), (1,)), ((), ())),
            preferred_element_type=jnp.float32)

        @pl.when(k == _K_CHUNKS - 1)
        def _():
            pred = acc_ref[...]
            t = t_ref[0]
            m = m_ref[0]
            c = jnp.log1p(jnp.exp(-jnp.abs(pred)))
            mn = jnp.minimum(pred, 0.0)
            ls_p = mn - c
            ls_mp = mn - pred - c
            bce = -(1.5 * t * ls_p + (1.0 - t) * ls_mp)
            o_ref[0, 0, 0] = jnp.sum(bce * m)
            o_ref[0, 0, 1] = jnp.sum(m) * float(win_sq)

    grid = (B, _K_CHUNKS)
    return pl.pallas_call(
        body,
        grid=grid,
        in_specs=[
            pl.BlockSpec((1, W, chunk), lambda b, k: (b, 0, k)),
            pl.BlockSpec((1, 1, N), lambda b, k: (b, 0, 0)),
            pl.BlockSpec((1, W, N), lambda b, k: (b, 0, 0)),
            pl.BlockSpec((1, 1, N), lambda b, k: (b, 0, 0)),
        ],
        out_specs=pl.BlockSpec((1, 1, 2), lambda b, k: (b, 0, 0),
                               memory_space=pltpu.SMEM),
        out_shape=jax.ShapeDtypeStruct((B, 1, 2), jnp.float32),
        scratch_shapes=[pltpu.VMEM((W, N), jnp.float32)],
    )(p_view, ind, target_t, maskf)


def kernel(output, mask, ind, target):
    B, S, W = output.shape
    N = ind.shape[1]
    win_sq = target.shape[-1] * target.shape[-2]
    p_view = output.transpose(0, 2, 1)
    target_t = target.reshape(B, N, win_sq).transpose(0, 2, 1)
    maskf = mask.astype(jnp.float32).reshape(B, 1, N)
    parts = _masked_bce_tc(p_view, ind.reshape(B, 1, N), target_t, maskf, win_sq)
    loss_sum = jnp.sum(parts[:, 0, 0])
    num_sample = jnp.sum(parts[:, 0, 1])
    return jnp.where(num_sample > 0, loss_sum / num_sample, loss_sum)
